# baseline (speedup 1.0000x reference)
import functools

import jax
import jax.numpy as jnp
import numpy as np
from jax import lax
from jax.experimental import pallas as pl
from jax.experimental.pallas import tpu as pltpu
from jax.experimental.pallas import tpu_sc as plsc

B = 4096
F = 26
V = 100000
D = 32
IN_DIM = F * D
PAD_DIM = 896
NU = PAD_DIM // 128
HID = 400
BF = B * F

NCHUNK = 1
BC = B // NCHUNK
BFC = BC * F

OUT_ROWS = BC * PAD_DIM // D
XP_ROWS = BC * PAD_DIM // 128


_NC = 2
_NS = 16
_NW = _NC * _NS
_ROWS_W = BFC // _NW


def _sc_gather(idx_hbm, oidx_hbm, table_hbm, out_hbm, idx_v, oidx_v, rows_v,
               sem):
    wid = lax.axis_index("s") * _NC + lax.axis_index("c")
    base = wid * _ROWS_W
    pltpu.sync_copy(idx_hbm.at[pl.ds(base, _ROWS_W)], idx_v)
    pltpu.sync_copy(oidx_hbm.at[pl.ds(base, _ROWS_W)], oidx_v)
    pltpu.async_copy(table_hbm.at[idx_v], rows_v, sem).wait()
    pltpu.async_copy(rows_v, out_hbm.at[oidx_v], sem).wait()


def _gather_rows(idx_flat, out_idx, table):
    mesh = plsc.VectorSubcoreMesh(core_axis_name="c", subcore_axis_name="s")
    k = functools.partial(
        pl.kernel,
        mesh=mesh,
        compiler_params=pltpu.CompilerParams(use_tc_tiling_on_sc=False),
        out_type=jax.ShapeDtypeStruct((OUT_ROWS, D), jnp.float32),
        scratch_types=[
            pltpu.VMEM((_ROWS_W,), jnp.int32),
            pltpu.VMEM((_ROWS_W,), jnp.int32),
            pltpu.VMEM((_ROWS_W, D), jnp.float32),
            pltpu.SemaphoreType.DMA,
        ],
    )(_sc_gather)
    return k(idx_flat, out_idx, table)


_BLK = 256
_NB = BC // _BLK
_TB = _BLK // 8
_XPB = _BLK * PAD_DIM // 128


_NCAT = HID + 4


def _dense_body(xp_ref, vals_ref, e_ref, wcat_ref, b1_ref, w2_ref, b2_ref,
                w3_ref, b3_ref, wdh_ref, t_ref, out_ref):
    f32 = jnp.float32
    bf16 = jnp.bfloat16
    xb = xp_ref[...].reshape(_TB, NU, 8, 128)
    scale = jnp.dot(vals_ref[...], e_ref[...], preferred_element_type=f32)
    lane = lax.broadcasted_iota(jnp.int32, (_BLK, 128), 1)
    hm = jnp.zeros((_BLK, _NCAT), f32)
    for u in range(NU):
        g = xb[:, u].reshape(_BLK, 128)
        if u == NU - 1:
            g = jnp.where(lane < 64, g, 0.0)
        x0u = g * scale[:, u * 128:(u + 1) * 128]
        hm = hm + jnp.dot(x0u.astype(bf16), wcat_ref[u * 128:(u + 1) * 128, :],
                          preferred_element_type=f32)
    h = jnp.maximum(hm[:, :HID] + b1_ref[...], 0.0)
    h = jnp.maximum(jnp.dot(h.astype(bf16), w2_ref[...],
                            preferred_element_type=f32) + b2_ref[...], 0.0)
    h = jnp.maximum(jnp.dot(h.astype(bf16), w3_ref[...],
                            preferred_element_type=f32) + b3_ref[...], 0.0)
    t = t_ref[...]
    a = 1.0 + hm[:, HID:HID + 1]
    a = a * (1.0 + hm[:, HID + 1:HID + 2]) + t[0, 0]
    a = a * (1.0 + hm[:, HID + 2:HID + 3]) + t[0, 1]
    logits = (jnp.dot(h, wdh_ref[...], preferred_element_type=f32)
              + a * hm[:, HID + 3:HID + 4] + t[0, 2])
    out_ref[...] = jax.nn.sigmoid(logits)


def _dense(xp, vals, expand, Wcat, b1, W2, b2, W3, b3, Wdh, tvec):
    full2 = lambda shape: pl.BlockSpec(shape, lambda i: (0, 0))
    return pl.pallas_call(
        _dense_body,
        grid=(_NB,),
        in_specs=[
            pl.BlockSpec((_XPB, 128), lambda i: (i, 0)),
            pl.BlockSpec((_BLK, F), lambda i: (i, 0)),
            full2((F, PAD_DIM)),
            full2((PAD_DIM, _NCAT)),
            full2((1, HID)),
            full2((HID, HID)),
            full2((1, HID)),
            full2((HID, HID)),
            full2((1, HID)),
            full2((HID, 1)),
            full2((1, 4)),
        ],
        out_specs=pl.BlockSpec((_BLK, 1), lambda i: (i, 0)),
        out_shape=jax.ShapeDtypeStruct((BC, 1), jnp.float32),
    )(xp, vals, expand, Wcat, b1, W2, b2, W3, b3, Wdh, tvec)


def kernel(feature_idx, feature_vals, feature_embedding, W1, b1, W2, b2, W3,
           b3, cw0, cb0, cw1, cb1, cw2, cb2, Wd, bd):
    idx_flat = feature_idx.reshape(BF)
    p = np.arange(BFC, dtype=np.int64)
    b_, f_ = p // F, p % F
    out_idx = jnp.asarray(
        ((b_ // 8) * NU + f_ // 4) * 32 + (b_ % 8) * 4 + (f_ % 4),
        dtype=jnp.int32)
    e_np = np.zeros((F, PAD_DIM), dtype=np.float32)
    for f in range(F):
        e_np[f, f * D:(f + 1) * D] = 1.0
    expand = jnp.asarray(e_np)
    Wdx = Wd[HID:]
    wcat = jnp.concatenate(
        [W1, cw0[:, None], cw1[:, None], cw2[:, None], Wdx], axis=1)
    wcat = jnp.concatenate(
        [wcat, jnp.zeros((PAD_DIM - IN_DIM, _NCAT), jnp.float32)],
        axis=0).astype(jnp.bfloat16)
    t1 = jnp.dot(cb0, cw1)
    t2 = jnp.dot(cb0 + cb1, cw2)
    c3 = jnp.dot(cb0 + cb1 + cb2, Wdx[:, 0]) + bd[0]
    tvec = jnp.stack([t1, t2, c3, jnp.float32(0)]).reshape(1, 4)
    outs = []
    for c in range(NCHUNK):
        idx_c = lax.slice(idx_flat, (c * BFC,), ((c + 1) * BFC,))
        gathered = _gather_rows(idx_c, out_idx, feature_embedding)
        xp = gathered.reshape(XP_ROWS, 128)
        vals_c = lax.slice(feature_vals, (c * BC, 0), ((c + 1) * BC, F))
        outs.append(_dense(xp, vals_c, expand, wcat, b1.reshape(1, HID),
                           W2.astype(jnp.bfloat16), b2.reshape(1, HID),
                           W3.astype(jnp.bfloat16), b3.reshape(1, HID),
                           Wd[:HID], tvec))
    if NCHUNK == 1:
        return outs[0]
    return jnp.concatenate(outs, axis=0)


# speedup vs baseline: 1.0491x; 1.0491x over previous
import functools

import jax
import jax.numpy as jnp
import numpy as np
from jax import lax
from jax.experimental import pallas as pl
from jax.experimental.pallas import tpu as pltpu
from jax.experimental.pallas import tpu_sc as plsc

B = 4096
F = 26
V = 100000
D = 32
IN_DIM = F * D
PAD_DIM = 896
NU = PAD_DIM // 128
HID = 400
BF = B * F

NCHUNK = 1
BC = B // NCHUNK
BFC = BC * F

OUT_ROWS = BC * PAD_DIM // D
XP_ROWS = BC * PAD_DIM // 128


_NC = 2
_NS = 16
_NW = _NC * _NS
_ROWS_W = BFC // _NW


_NSUB = 2
_RSUB = _ROWS_W // _NSUB


def _sc_gather(idx_hbm, oidx_hbm, table_hbm, out_hbm, idx_v, oidx_v, rows_v,
               gsem, ssem):
    wid = lax.axis_index("s") * _NC + lax.axis_index("c")
    base = wid * _ROWS_W
    pltpu.sync_copy(idx_hbm.at[pl.ds(base, _ROWS_W)], idx_v)
    for c in range(_NSUB):
        pltpu.sync_copy(oidx_hbm.at[pl.ds(base + c * _RSUB, _RSUB)],
                        oidx_v.at[c])
    gathers = []
    for c in range(_NSUB):
        gathers.append(pltpu.async_copy(
            table_hbm.at[idx_v.at[pl.ds(c * _RSUB, _RSUB)]],
            rows_v.at[c], gsem))
    scatters = []
    for c in range(_NSUB):
        gathers[c].wait()
        scatters.append(pltpu.async_copy(
            rows_v.at[c], out_hbm.at[oidx_v.at[c]], ssem))
    for s in scatters:
        s.wait()


def _gather_rows(idx_flat, out_idx, table):
    mesh = plsc.VectorSubcoreMesh(core_axis_name="c", subcore_axis_name="s")
    k = functools.partial(
        pl.kernel,
        mesh=mesh,
        compiler_params=pltpu.CompilerParams(use_tc_tiling_on_sc=False),
        out_type=jax.ShapeDtypeStruct((OUT_ROWS, D), jnp.float32),
        scratch_types=[
            pltpu.VMEM((_ROWS_W,), jnp.int32),
            pltpu.VMEM((_NSUB, _RSUB), jnp.int32),
            pltpu.VMEM((_NSUB, _RSUB, D), jnp.float32),
            pltpu.SemaphoreType.DMA,
            pltpu.SemaphoreType.DMA,
        ],
    )(_sc_gather)
    return k(idx_flat, out_idx, table)


_BLK = 512
_NB = BC // _BLK
_TB = _BLK // 8
_XPB = _BLK * PAD_DIM // 128


_NCAT = HID + 4


def _dense_body(xp_ref, vals_ref, e_ref, wcat_ref, b1_ref, w2_ref, b2_ref,
                w3_ref, b3_ref, wdh_ref, t_ref, out_ref):
    f32 = jnp.float32
    bf16 = jnp.bfloat16
    xb = xp_ref[...].reshape(_TB, NU, 8, 128)
    scale = jnp.dot(vals_ref[...], e_ref[...], preferred_element_type=f32)
    lane = lax.broadcasted_iota(jnp.int32, (_BLK, 128), 1)
    hm = jnp.zeros((_BLK, _NCAT), f32)
    for u in range(NU):
        g = xb[:, u].reshape(_BLK, 128)
        if u == NU - 1:
            g = jnp.where(lane < 64, g, 0.0)
        x0u = g * scale[:, u * 128:(u + 1) * 128]
        hm = hm + jnp.dot(x0u.astype(bf16), wcat_ref[u * 128:(u + 1) * 128, :],
                          preferred_element_type=f32)
    h = jnp.maximum(hm[:, :HID] + b1_ref[...], 0.0)
    h = jnp.maximum(jnp.dot(h.astype(bf16), w2_ref[...],
                            preferred_element_type=f32) + b2_ref[...], 0.0)
    h = jnp.maximum(jnp.dot(h.astype(bf16), w3_ref[...],
                            preferred_element_type=f32) + b3_ref[...], 0.0)
    t = t_ref[...]
    a = 1.0 + hm[:, HID:HID + 1]
    a = a * (1.0 + hm[:, HID + 1:HID + 2]) + t[0, 0]
    a = a * (1.0 + hm[:, HID + 2:HID + 3]) + t[0, 1]
    logits = (jnp.dot(h, wdh_ref[...], preferred_element_type=f32)
              + a * hm[:, HID + 3:HID + 4] + t[0, 2])
    out_ref[...] = jax.nn.sigmoid(logits)


def _dense(xp, vals, expand, Wcat, b1, W2, b2, W3, b3, Wdh, tvec):
    full2 = lambda shape: pl.BlockSpec(shape, lambda i: (0, 0))
    return pl.pallas_call(
        _dense_body,
        grid=(_NB,),
        in_specs=[
            pl.BlockSpec((_XPB, 128), lambda i: (i, 0)),
            pl.BlockSpec((_BLK, F), lambda i: (i, 0)),
            full2((F, PAD_DIM)),
            full2((PAD_DIM, _NCAT)),
            full2((1, HID)),
            full2((HID, HID)),
            full2((1, HID)),
            full2((HID, HID)),
            full2((1, HID)),
            full2((HID, 1)),
            full2((1, 4)),
        ],
        out_specs=pl.BlockSpec((_BLK, 1), lambda i: (i, 0)),
        out_shape=jax.ShapeDtypeStruct((BC, 1), jnp.float32),
    )(xp, vals, expand, Wcat, b1, W2, b2, W3, b3, Wdh, tvec)


def kernel(feature_idx, feature_vals, feature_embedding, W1, b1, W2, b2, W3,
           b3, cw0, cb0, cw1, cb1, cw2, cb2, Wd, bd):
    idx_flat = feature_idx.reshape(BF)
    p = np.arange(BFC, dtype=np.int64)
    b_, f_ = p // F, p % F
    out_idx = jnp.asarray(
        ((b_ // 8) * NU + f_ // 4) * 32 + (b_ % 8) * 4 + (f_ % 4),
        dtype=jnp.int32)
    e_np = np.zeros((F, PAD_DIM), dtype=np.float32)
    for f in range(F):
        e_np[f, f * D:(f + 1) * D] = 1.0
    expand = jnp.asarray(e_np)
    Wdx = Wd[HID:]
    wcat = jnp.concatenate(
        [W1, cw0[:, None], cw1[:, None], cw2[:, None], Wdx], axis=1)
    wcat = jnp.concatenate(
        [wcat, jnp.zeros((PAD_DIM - IN_DIM, _NCAT), jnp.float32)],
        axis=0).astype(jnp.bfloat16)
    t1 = jnp.dot(cb0, cw1)
    t2 = jnp.dot(cb0 + cb1, cw2)
    c3 = jnp.dot(cb0 + cb1 + cb2, Wdx[:, 0]) + bd[0]
    tvec = jnp.stack([t1, t2, c3, jnp.float32(0)]).reshape(1, 4)
    outs = []
    for c in range(NCHUNK):
        idx_c = lax.slice(idx_flat, (c * BFC,), ((c + 1) * BFC,))
        gathered = _gather_rows(idx_c, out_idx, feature_embedding)
        xp = gathered.reshape(XP_ROWS, 128)
        vals_c = lax.slice(feature_vals, (c * BC, 0), ((c + 1) * BC, F))
        outs.append(_dense(xp, vals_c, expand, wcat, b1.reshape(1, HID),
                           W2.astype(jnp.bfloat16), b2.reshape(1, HID),
                           W3.astype(jnp.bfloat16), b3.reshape(1, HID),
                           Wd[:HID], tvec))
    if NCHUNK == 1:
        return outs[0]
    return jnp.concatenate(outs, axis=0)


# speedup vs baseline: 1.0580x; 1.0085x over previous
import functools

import jax
import jax.numpy as jnp
import numpy as np
from jax import lax
from jax.experimental import pallas as pl
from jax.experimental.pallas import tpu as pltpu
from jax.experimental.pallas import tpu_sc as plsc

B = 4096
F = 26
V = 100000
D = 32
IN_DIM = F * D
PAD_DIM = 896
NU = PAD_DIM // 128
HID = 400
BF = B * F

NCHUNK = 1
BC = B // NCHUNK
BFC = BC * F

OUT_ROWS = BC * PAD_DIM // D
XP_ROWS = BC * PAD_DIM // 128


_NC = 2
_NS = 16
_NW = _NC * _NS
_ROWS_W = BFC // _NW


def _sc_gather(idx_hbm, oidx_hbm, table_hbm, out_hbm, idx_v, oidx_v, rows_v,
               sem):
    wid = lax.axis_index("s") * _NC + lax.axis_index("c")
    base = wid * _ROWS_W
    pltpu.sync_copy(idx_hbm.at[pl.ds(base, _ROWS_W)], idx_v)
    pltpu.sync_copy(oidx_hbm.at[pl.ds(base, _ROWS_W)], oidx_v)
    pltpu.async_copy(table_hbm.at[idx_v], rows_v, sem).wait()
    pltpu.async_copy(rows_v, out_hbm.at[oidx_v], sem).wait()


def _gather_rows(idx_flat, out_idx, table):
    mesh = plsc.VectorSubcoreMesh(core_axis_name="c", subcore_axis_name="s")
    k = functools.partial(
        pl.kernel,
        mesh=mesh,
        compiler_params=pltpu.CompilerParams(use_tc_tiling_on_sc=False),
        out_type=jax.ShapeDtypeStruct((OUT_ROWS, D), jnp.float32),
        scratch_types=[
            pltpu.VMEM((_ROWS_W,), jnp.int32),
            pltpu.VMEM((_ROWS_W,), jnp.int32),
            pltpu.VMEM((_ROWS_W, D), jnp.float32),
            pltpu.SemaphoreType.DMA,
        ],
    )(_sc_gather)
    return k(idx_flat, out_idx, table)


_BLK = 512
_NB = BC // _BLK
_TB = _BLK // 8
_XPB = _BLK * PAD_DIM // 128


_NCAT = HID + 4


def _dense_body(xp_ref, vals_ref, e_ref, wcat_ref, b1_ref, w2_ref, b2_ref,
                w3_ref, b3_ref, wdh_ref, t_ref, out_ref):
    f32 = jnp.float32
    bf16 = jnp.bfloat16
    xb = xp_ref[...].reshape(_TB, NU, 8, 128)
    scale = jnp.dot(vals_ref[...], e_ref[...], preferred_element_type=f32)
    lane = lax.broadcasted_iota(jnp.int32, (_BLK, 128), 1)
    hm = jnp.zeros((_BLK, _NCAT), f32)
    for u in range(NU):
        g = xb[:, u].reshape(_BLK, 128)
        if u == NU - 1:
            g = jnp.where(lane < 64, g, 0.0)
        x0u = g * scale[:, u * 128:(u + 1) * 128]
        hm = hm + jnp.dot(x0u.astype(bf16), wcat_ref[u * 128:(u + 1) * 128, :],
                          preferred_element_type=f32)
    h = jnp.maximum(hm[:, :HID] + b1_ref[...], 0.0)
    h = jnp.maximum(jnp.dot(h.astype(bf16), w2_ref[...],
                            preferred_element_type=f32) + b2_ref[...], 0.0)
    h = jnp.maximum(jnp.dot(h.astype(bf16), w3_ref[...],
                            preferred_element_type=f32) + b3_ref[...], 0.0)
    t = t_ref[...]
    a = 1.0 + hm[:, HID:HID + 1]
    a = a * (1.0 + hm[:, HID + 1:HID + 2]) + t[0, 0]
    a = a * (1.0 + hm[:, HID + 2:HID + 3]) + t[0, 1]
    logits = (jnp.dot(h, wdh_ref[...], preferred_element_type=f32)
              + a * hm[:, HID + 3:HID + 4] + t[0, 2])
    out_ref[...] = jax.nn.sigmoid(logits)


def _dense(xp, vals, expand, Wcat, b1, W2, b2, W3, b3, Wdh, tvec):
    full2 = lambda shape: pl.BlockSpec(shape, lambda i: (0, 0))
    return pl.pallas_call(
        _dense_body,
        grid=(_NB,),
        in_specs=[
            pl.BlockSpec((_XPB, 128), lambda i: (i, 0)),
            pl.BlockSpec((_BLK, F), lambda i: (i, 0)),
            full2((F, PAD_DIM)),
            full2((PAD_DIM, _NCAT)),
            full2((1, HID)),
            full2((HID, HID)),
            full2((1, HID)),
            full2((HID, HID)),
            full2((1, HID)),
            full2((HID, 1)),
            full2((1, 4)),
        ],
        out_specs=pl.BlockSpec((_BLK, 1), lambda i: (i, 0)),
        out_shape=jax.ShapeDtypeStruct((BC, 1), jnp.float32),
    )(xp, vals, expand, Wcat, b1, W2, b2, W3, b3, Wdh, tvec)


def kernel(feature_idx, feature_vals, feature_embedding, W1, b1, W2, b2, W3,
           b3, cw0, cb0, cw1, cb1, cw2, cb2, Wd, bd):
    idx_flat = feature_idx.reshape(BF)
    p = np.arange(BFC, dtype=np.int64)
    b_, f_ = p // F, p % F
    out_idx = jnp.asarray(
        ((b_ // 8) * NU + f_ // 4) * 32 + (b_ % 8) * 4 + (f_ % 4),
        dtype=jnp.int32)
    e_np = np.zeros((F, PAD_DIM), dtype=np.float32)
    for f in range(F):
        e_np[f, f * D:(f + 1) * D] = 1.0
    expand = jnp.asarray(e_np)
    Wdx = Wd[HID:]
    wcat = jnp.concatenate(
        [W1, cw0[:, None], cw1[:, None], cw2[:, None], Wdx], axis=1)
    wcat = jnp.concatenate(
        [wcat, jnp.zeros((PAD_DIM - IN_DIM, _NCAT), jnp.float32)],
        axis=0).astype(jnp.bfloat16)
    t1 = jnp.dot(cb0, cw1)
    t2 = jnp.dot(cb0 + cb1, cw2)
    c3 = jnp.dot(cb0 + cb1 + cb2, Wdx[:, 0]) + bd[0]
    tvec = jnp.stack([t1, t2, c3, jnp.float32(0)]).reshape(1, 4)
    outs = []
    for c in range(NCHUNK):
        idx_c = lax.slice(idx_flat, (c * BFC,), ((c + 1) * BFC,))
        gathered = _gather_rows(idx_c, out_idx, feature_embedding)
        xp = gathered.reshape(XP_ROWS, 128)
        vals_c = lax.slice(feature_vals, (c * BC, 0), ((c + 1) * BC, F))
        outs.append(_dense(xp, vals_c, expand, wcat, b1.reshape(1, HID),
                           W2.astype(jnp.bfloat16), b2.reshape(1, HID),
                           W3.astype(jnp.bfloat16), b3.reshape(1, HID),
                           Wd[:HID], tvec))
    if NCHUNK == 1:
        return outs[0]
    return jnp.concatenate(outs, axis=0)
